# baseline (speedup 1.0000x reference)
import functools

import jax
import jax.numpy as jnp
from jax import lax
from jax.experimental import pallas as pl
from jax.experimental.pallas import tpu as pltpu
from jax.experimental.pallas import tpu_sc as plsc

NC = 2
NS = 16
NW = NC * NS

BATCH = 16384
F = 26
D = 64
V = 1000000
BPT = BATCH // NW
BHI = BPT // 128
HB = BPT // 2


def _body(x_hbm, idx_hbm, tab_hbm, out_hbm, idxv, rb0, rb1, xb0, xb1,
          ob0, ob1, gsem, xsem, osem):
    wid = lax.axis_index("s") * NC + lax.axis_index("c")
    rbs = (rb0, rb1)
    xbs = (xb0, xb1)
    obs = (ob0, ob1)

    pltpu.sync_copy(idx_hbm.at[:, pl.ds(wid * BPT, BPT)], idxv)

    def issue_gather(u, p):
        f, h = u // 2, u % 2
        pltpu.async_copy(
            tab_hbm.at[idxv.at[f, pl.ds(h * HB, HB)]], rbs[p], gsem)

    def issue_x(u, p):
        f, h = u // 2, u % 2
        pltpu.async_copy(
            x_hbm.at[f, :, pl.ds(wid * BHI + h * 2, 2), :, :], xbs[p], xsem)

    def issue_out(u, p):
        f, h = u // 2, u % 2
        pltpu.async_copy(
            obs[p], out_hbm.at[f, :, pl.ds(wid * BHI + h * 2, 2), :, :], osem)

    def wait(sem, ref):
        if ref.shape == (HB, D):
            src = tab_hbm.at[pl.ds(0, HB), :]
        else:
            src = x_hbm.at[0, :, pl.ds(0, 2), :, :]
        pltpu.make_async_copy(src, ref, sem).wait()

    iota16 = lax.iota(jnp.int32, 16)

    issue_gather(0, 0)
    issue_x(0, 0)

    NU = 2 * F
    for u in range(NU):
        p = u % 2
        rb_p, xb_p, ob_p = rbs[p], xbs[p], obs[p]
        wait(gsem, rb_p)
        wait(xsem, xb_p)
        if u + 1 < NU:
            issue_gather(u + 1, 1 - p)
            issue_x(u + 1, 1 - p)

        def compute(t, carry):
            dhi = t >> 7
            j2 = (t >> 6) & 1
            dlo = (t >> 3) & 7
            bq = t & 7
            bvec = iota16 + (j2 * 128 + bq * 16)
            dvec = jnp.full((16,), 0, jnp.int32) + (dhi * 8 + dlo)
            val = plsc.load_gather(rb_p, [bvec, dvec])
            s = pl.ds(bq * 16, 16)
            ob_p[dhi, j2, dlo, s] = xb_p[dhi, j2, dlo, s] + val
            return carry

        lax.fori_loop(0, 1024, compute, 0, unroll=4)
        issue_out(u, p)
        if u >= 1:
            wait(osem, obs[1 - p])
    wait(osem, obs[1])


_sc_call = functools.partial(
    pl.kernel,
    mesh=plsc.VectorSubcoreMesh(core_axis_name="c", subcore_axis_name="s"),
    out_type=jax.ShapeDtypeStruct((F, 8, 128, 8, 128), jnp.float32),
    scratch_types=[
        pltpu.VMEM((F, BPT), jnp.int32),
        pltpu.VMEM((HB, D), jnp.float32),
        pltpu.VMEM((HB, D), jnp.float32),
        pltpu.VMEM((8, 2, 8, 128), jnp.float32),
        pltpu.VMEM((8, 2, 8, 128), jnp.float32),
        pltpu.VMEM((8, 2, 8, 128), jnp.float32),
        pltpu.VMEM((8, 2, 8, 128), jnp.float32),
        pltpu.SemaphoreType.DMA,
        pltpu.SemaphoreType.DMA,
        pltpu.SemaphoreType.DMA,
    ],
    compiler_params=pltpu.CompilerParams(use_tc_tiling_on_sc=False,
                                         needs_layout_passes=False),
)(_body)


@jax.jit
def kernel(x, idxs, embed_weight):
    x5 = (x.transpose(1, 2, 0)
           .reshape(F, 8, 8, 128, 128)
           .transpose(0, 1, 3, 2, 4))
    idxT = idxs.astype(jnp.int32).T
    out5 = _sc_call(x5, idxT, embed_weight)
    return (out5.transpose(0, 1, 3, 2, 4)
                .reshape(F, D, BATCH)
                .transpose(2, 0, 1))


# speedup vs baseline: 1.1872x; 1.1872x over previous
import functools

import jax
import jax.numpy as jnp
from jax import lax
from jax.experimental import pallas as pl
from jax.experimental.pallas import tpu as pltpu
from jax.experimental.pallas import tpu_sc as plsc

NC = 2
NS = 16
NW = NC * NS

BATCH = 16384
F = 26
D = 64
V = 1000000
BPT = BATCH // NW
NBUF = 3


def _body(x_hbm, idx_hbm, tab_hbm, out_hbm, idx_v, buf0, buf1, buf2,
          xsem, gsem, osem):
    wid = lax.axis_index("s") * NC + lax.axis_index("c")
    b0 = wid * BPT
    bufs = (buf0, buf1, buf2)

    pltpu.sync_copy(idx_hbm.at[:, pl.ds(b0, BPT)], idx_v)

    def issue_x(f):
        pltpu.async_copy(
            x_hbm.at[f, pl.ds(b0, BPT), :], bufs[f % NBUF], xsem)

    def issue_gather(f):
        pltpu.async_copy(
            tab_hbm.at[idx_v.at[f]], bufs[f % NBUF], gsem, add=True)

    def issue_out(f):
        pltpu.async_copy(
            bufs[f % NBUF], out_hbm.at[f, pl.ds(b0, BPT), :], osem)

    def wait(sem):
        pltpu.make_async_copy(
            x_hbm.at[0, pl.ds(0, BPT), :], buf0, sem).wait()

    issue_x(0)
    issue_x(1)

    for f in range(F):
        wait(xsem)
        issue_gather(f)
        wait(gsem)
        issue_out(f)
        if f >= 1:
            wait(osem)
        if f + 2 < F:
            issue_x(f + 2)

    wait(osem)


_sc_call = functools.partial(
    pl.kernel,
    mesh=plsc.VectorSubcoreMesh(core_axis_name="c", subcore_axis_name="s"),
    out_type=jax.ShapeDtypeStruct((F, BATCH, D), jnp.float32),
    scratch_types=[
        pltpu.VMEM((F, BPT), jnp.int32),
        pltpu.VMEM((BPT, D), jnp.float32),
        pltpu.VMEM((BPT, D), jnp.float32),
        pltpu.VMEM((BPT, D), jnp.float32),
        pltpu.SemaphoreType.DMA,
        pltpu.SemaphoreType.DMA,
        pltpu.SemaphoreType.DMA,
    ],
    compiler_params=pltpu.CompilerParams(use_tc_tiling_on_sc=False),
)(_body)


@jax.jit
def kernel(x, idxs, embed_weight):
    xT = x.transpose(1, 0, 2)
    idxT = idxs.astype(jnp.int32).T
    outT = _sc_call(xT, idxT, embed_weight)
    return outT.transpose(1, 0, 2)
